# baseline (speedup 1.0000x reference)
import functools

import jax
import jax.numpy as jnp
from jax.experimental import pallas as pl
from jax.experimental.pallas import tpu as pltpu

_P = 256
_D = 768
_V = 49408
_VB = 8192


def _argmin_kernel(a_ref, c_ref, ids_ref, minval_ref, minidx_ref, *, n_blocks):
    j = pl.program_id(0)

    @pl.when(j == 0)
    def _init():
        minval_ref[...] = jnp.full((_P, 1), jnp.inf, dtype=jnp.float32)
        minidx_ref[...] = jnp.zeros((_P, 1), dtype=jnp.int32)

    a = a_ref[...]
    c = c_ref[...]
    a2 = jnp.sum(a * a, axis=1, keepdims=True)
    b2 = jnp.sum(c * c, axis=1, keepdims=True)
    s = jax.lax.dot_general(
        a, c, (((1,), (1,)), ((), ())),
        preferred_element_type=jnp.float32,
        precision=jax.lax.Precision.DEFAULT,
    )
    d2 = (a2 + b2.T) - 2.0 * s

    col = j * _VB + jax.lax.broadcasted_iota(jnp.int32, (_P, _VB), 1)
    d2 = jnp.where(col < _V, d2, jnp.inf)

    bmin = jnp.min(d2, axis=1, keepdims=True)
    bidx = jnp.argmin(d2, axis=1).astype(jnp.int32)
    bidx = bidx.reshape(_P, 1) + j * _VB

    upd = bmin < minval_ref[...]
    minidx_ref[...] = jnp.where(upd, bidx, minidx_ref[...])
    minval_ref[...] = jnp.where(upd, bmin, minval_ref[...])

    @pl.when(j == n_blocks - 1)
    def _done():
        ids_ref[...] = minidx_ref[...]


def _nearest_ids(prompt_embs, clip_embs):
    n_blocks = pl.cdiv(_V, _VB)
    ids = pl.pallas_call(
        functools.partial(_argmin_kernel, n_blocks=n_blocks),
        grid=(n_blocks,),
        in_specs=[
            pl.BlockSpec((_P, _D), lambda j: (0, 0)),
            pl.BlockSpec((_VB, _D), lambda j: (j, 0)),
        ],
        out_specs=pl.BlockSpec((_P, 1), lambda j: (0, 0)),
        out_shape=jax.ShapeDtypeStruct((_P, 1), jnp.int32),
        scratch_shapes=[
            pltpu.VMEM((_P, 1), jnp.float32),
            pltpu.VMEM((_P, 1), jnp.int32),
        ],
        compiler_params=pltpu.CompilerParams(
            dimension_semantics=("arbitrary",),
        ),
    )(prompt_embs, clip_embs)
    return ids.reshape(_P)


@jax.jit
def kernel(prompt_embs, clip_embs):
    ids = _nearest_ids(prompt_embs, clip_embs)
    return (prompt_embs, prompt_embs, ids)


# speedup vs baseline: 1.0845x; 1.0845x over previous
import functools

import jax
import jax.numpy as jnp
from jax.experimental import pallas as pl
from jax.experimental.pallas import tpu as pltpu

_P = 256
_D = 768
_V = 49408
_VB = 3088


def _argmin_kernel(a_ref, c_ref, ids_ref, minval_ref, minidx_ref, *, n_blocks):
    j = pl.program_id(0)

    @pl.when(j == 0)
    def _init():
        minval_ref[...] = jnp.full((_P, 1), jnp.inf, dtype=jnp.float32)
        minidx_ref[...] = jnp.zeros((_P, 1), dtype=jnp.int32)

    a = a_ref[...]
    c = c_ref[...]
    a2 = jnp.sum(a * a, axis=1, keepdims=True)
    b2 = jnp.sum(c * c, axis=1, keepdims=True)
    s = jax.lax.dot_general(
        a, c, (((1,), (1,)), ((), ())),
        preferred_element_type=jnp.float32,
        precision=jax.lax.Precision.DEFAULT,
    )
    d2 = (a2 + b2.T) - 2.0 * s

    col = j * _VB + jax.lax.broadcasted_iota(jnp.int32, (_P, _VB), 1)
    d2 = jnp.where(col < _V, d2, jnp.inf)

    bmin = jnp.min(d2, axis=1, keepdims=True)
    bidx = jnp.argmin(d2, axis=1).astype(jnp.int32)
    bidx = bidx.reshape(_P, 1) + j * _VB

    upd = bmin < minval_ref[...]
    minidx_ref[...] = jnp.where(upd, bidx, minidx_ref[...])
    minval_ref[...] = jnp.where(upd, bmin, minval_ref[...])

    @pl.when(j == n_blocks - 1)
    def _done():
        ids_ref[...] = minidx_ref[...]


def _nearest_ids(prompt_embs, clip_embs):
    n_blocks = pl.cdiv(_V, _VB)
    ids = pl.pallas_call(
        functools.partial(_argmin_kernel, n_blocks=n_blocks),
        grid=(n_blocks,),
        in_specs=[
            pl.BlockSpec((_P, _D), lambda j: (0, 0)),
            pl.BlockSpec((_VB, _D), lambda j: (j, 0)),
        ],
        out_specs=pl.BlockSpec((_P, 1), lambda j: (0, 0)),
        out_shape=jax.ShapeDtypeStruct((_P, 1), jnp.int32),
        scratch_shapes=[
            pltpu.VMEM((_P, 1), jnp.float32),
            pltpu.VMEM((_P, 1), jnp.int32),
        ],
        compiler_params=pltpu.CompilerParams(
            dimension_semantics=("arbitrary",),
        ),
    )(prompt_embs, clip_embs)
    return ids.reshape(_P)


@jax.jit
def kernel(prompt_embs, clip_embs):
    ids = _nearest_ids(prompt_embs, clip_embs)
    return (prompt_embs, prompt_embs, ids)


# speedup vs baseline: 1.1449x; 1.0557x over previous
import functools

import jax
import jax.numpy as jnp
from jax.experimental import pallas as pl
from jax.experimental.pallas import tpu as pltpu

_P = 256
_D = 768
_V = 49408
_VB = 6176


def _argmin_kernel(a_ref, c_ref, ids_ref, minval_ref, minidx_ref, *, n_blocks):
    j = pl.program_id(0)

    @pl.when(j == 0)
    def _init():
        minval_ref[...] = jnp.full((_P, 1), jnp.inf, dtype=jnp.float32)
        minidx_ref[...] = jnp.zeros((_P, 1), dtype=jnp.int32)

    a = a_ref[...]
    c = c_ref[...]
    a2 = jnp.sum(a * a, axis=1, keepdims=True)
    b2 = jnp.sum(c * c, axis=1, keepdims=True)
    s = jax.lax.dot_general(
        a, c, (((1,), (1,)), ((), ())),
        preferred_element_type=jnp.float32,
        precision=jax.lax.Precision.DEFAULT,
    )
    d2 = (a2 + b2.T) - 2.0 * s

    col = j * _VB + jax.lax.broadcasted_iota(jnp.int32, (_P, _VB), 1)
    d2 = jnp.where(col < _V, d2, jnp.inf)

    bmin = jnp.min(d2, axis=1, keepdims=True)
    bidx = jnp.argmin(d2, axis=1).astype(jnp.int32)
    bidx = bidx.reshape(_P, 1) + j * _VB

    upd = bmin < minval_ref[...]
    minidx_ref[...] = jnp.where(upd, bidx, minidx_ref[...])
    minval_ref[...] = jnp.where(upd, bmin, minval_ref[...])

    @pl.when(j == n_blocks - 1)
    def _done():
        ids_ref[...] = minidx_ref[...]


def _nearest_ids(prompt_embs, clip_embs):
    n_blocks = pl.cdiv(_V, _VB)
    ids = pl.pallas_call(
        functools.partial(_argmin_kernel, n_blocks=n_blocks),
        grid=(n_blocks,),
        in_specs=[
            pl.BlockSpec((_P, _D), lambda j: (0, 0)),
            pl.BlockSpec((_VB, _D), lambda j: (j, 0)),
        ],
        out_specs=pl.BlockSpec((_P, 1), lambda j: (0, 0)),
        out_shape=jax.ShapeDtypeStruct((_P, 1), jnp.int32),
        scratch_shapes=[
            pltpu.VMEM((_P, 1), jnp.float32),
            pltpu.VMEM((_P, 1), jnp.int32),
        ],
        compiler_params=pltpu.CompilerParams(
            dimension_semantics=("arbitrary",),
        ),
    )(prompt_embs, clip_embs)
    return ids.reshape(_P)


@jax.jit
def kernel(prompt_embs, clip_embs):
    ids = _nearest_ids(prompt_embs, clip_embs)
    return (prompt_embs, prompt_embs, ids)
